# baseline (speedup 1.0000x reference)
import functools
import math

import jax
import jax.numpy as jnp
from jax import lax
from jax.experimental import pallas as pl
from jax.experimental.pallas import tpu as pltpu
from jax.experimental.pallas import tpu_sc as plsc

N0 = 10000
E0 = 160000
HID = 256
DEPTH = 3
RATIO = 0.5
EDGE_LEN = E0 + N0
K_EDGE = 256
N_TILE = 16
_pt = -(-EDGE_LEN // N_TILE)
NCH = -(-_pt // K_EDGE)
EDGE_PAD = N_TILE * NCH * K_EDGE


def _row_chunks(total):
    out = []
    left = total
    while left > 0:
        sz = min(K_EDGE, left)
        out.append(sz)
        left -= sz
    return out


def _mm_body(x_ref, w_ref, d_ref, o_ref):
    o_ref[...] = (
        jax.lax.dot_general(
            x_ref[...], w_ref[...],
            (((1,), (0,)), ((), ())),
            precision=jax.lax.Precision.DEFAULT,
            preferred_element_type=jnp.float32)
        * d_ref[...]
    )


@functools.partial(jax.jit, static_argnames=("bm",))
def _matmul_scaled(x, W, dinv, bm=256):
    n, cin = x.shape
    cout = W.shape[1]
    n_pad = -(-n // bm) * bm
    xp = jnp.pad(x, ((0, n_pad - n), (0, 0)))
    dp = jnp.pad(dinv, ((0, n_pad - n),))[:, None]
    out = pl.pallas_call(
        _mm_body,
        grid=(n_pad // bm,),
        in_specs=[
            pl.BlockSpec((bm, cin), lambda i: (i, 0)),
            pl.BlockSpec((cin, cout), lambda i: (0, 0)),
            pl.BlockSpec((bm, 1), lambda i: (i, 0)),
        ],
        out_specs=pl.BlockSpec((bm, cout), lambda i: (i, 0)),
        out_shape=jax.ShapeDtypeStruct((n_pad, cout), jnp.float32),
    )(xp, W, dp)
    return out[:n]


@functools.partial(jax.jit, static_argnames=("n",))
def _scatter_agg(h_lo, h_hi, c_arr, r_arr, n):
    n_out = 128 * (-(-n // 128))
    rows_pt = n_out // N_TILE
    mesh = plsc.VectorSubcoreMesh(core_axis_name="c", subcore_axis_name="s")

    @functools.partial(
        pl.kernel,
        mesh=mesh,
        out_type=[jax.ShapeDtypeStruct((n_out, 128), jnp.float32)] * 2,
        scratch_types=[
            pltpu.VMEM((K_EDGE,), jnp.int32),
            pltpu.VMEM((K_EDGE,), jnp.int32),
            pltpu.VMEM((K_EDGE, 128), jnp.float32),
            pltpu.VMEM_SHARED((n_out, 128), jnp.float32),
            pltpu.SemaphoreType.DMA,
        ],
    )
    def k(hlo, hhi, c_hbm, r_hbm, z_hbm, out_lo, out_hi,
          c_v, r_v, rows_v, acc_sh, sem):
        cid = lax.axis_index("c")
        sid = lax.axis_index("s")
        base = sid * rows_pt

        off = 0
        for sz in _row_chunks(rows_pt):
            pltpu.sync_copy(z_hbm.at[pl.ds(0, sz)],
                            acc_sh.at[pl.ds(base + off, sz)])
            off += sz
        plsc.subcore_barrier()

        def run(h_hbm, out_hbm):
            def chunk(j, carry):
                e0 = (sid * NCH + j) * K_EDGE
                pltpu.sync_copy(c_hbm.at[pl.ds(e0, K_EDGE)], c_v)
                pltpu.sync_copy(r_hbm.at[pl.ds(e0, K_EDGE)], r_v)
                pltpu.async_copy(h_hbm.at[c_v], rows_v, sem).wait()
                pltpu.sync_copy(rows_v, acc_sh.at[r_v], add=True)
                return carry

            lax.fori_loop(0, NCH, chunk, 0)
            plsc.subcore_barrier()
            off2 = 0
            for sz in _row_chunks(rows_pt):
                pltpu.sync_copy(acc_sh.at[pl.ds(base + off2, sz)],
                                rows_v.at[pl.ds(0, sz)])
                pltpu.sync_copy(rows_v.at[pl.ds(0, sz)],
                                out_hbm.at[pl.ds(base + off2, sz)])
                off2 += sz

        @pl.when(cid == 0)
        def _():
            run(hlo, out_lo)

        @pl.when(cid == 1)
        def _():
            run(hhi, out_hi)

    zeros = jnp.zeros((K_EDGE, 128), jnp.float32)
    return k(h_lo, h_hi, c_arr, r_arr, zeros)


def _gcn_ref(x, row, col, ew, W, b, n):
    sl = jnp.arange(n, dtype=row.dtype)
    r = jnp.concatenate([row, sl])
    c = jnp.concatenate([col, sl])
    w = jnp.concatenate([ew, 2.0 * jnp.ones((n,), jnp.float32)])
    deg = jax.ops.segment_sum(w, c, num_segments=n)
    safe = jnp.where(deg > 0, deg, 1.0)
    dinv = jnp.where(deg > 0, 1.0 / jnp.sqrt(safe), 0.0)
    norm = dinv[r] * w * dinv[c]
    h = _matmul_scaled(x, W, jnp.ones((n,), jnp.float32))
    return jax.ops.segment_sum(norm[:, None] * h[c], r, num_segments=n) + b


def _gcn_sc(x, row, col, ew, W, b, n):
    deg = jax.ops.segment_sum(ew, col, num_segments=n) + 2.0
    dinv = lax.rsqrt(deg)
    hp = _matmul_scaled(x, W, dinv)
    ce = jnp.where(ew > 0.0, col, n).astype(jnp.int32)
    re = jnp.where(ew > 0.0, row, 0).astype(jnp.int32)
    c_arr = jnp.pad(ce, (0, EDGE_PAD - EDGE_LEN), constant_values=n)
    r_arr = jnp.pad(re, (0, EDGE_PAD - EDGE_LEN))
    hpad = jnp.pad(hp, ((0, 8), (0, 0)))
    lo, hi = _scatter_agg(hpad[:, :128], hpad[:, 128:], c_arr, r_arr, n)
    agg = jnp.concatenate([lo[:n], hi[:n]], axis=1)
    return dinv[:, None] * (agg + 2.0 * hp) + b


def _pool(x, row, col, ew, batch, p, n, k):
    score = jnp.tanh((x @ p) / jnp.linalg.norm(p))
    vals, perm = lax.top_k(score, k)
    x_new = x[perm] * vals[:, None]
    batch_new = batch[perm]
    inv = jnp.full((n,), -1, dtype=row.dtype).at[perm].set(
        jnp.arange(k, dtype=row.dtype))
    r2 = inv[row]
    c2 = inv[col]
    mask = (r2 >= 0) & (c2 >= 0)
    row_new = jnp.where(mask, r2, 0)
    col_new = jnp.where(mask, c2, 0)
    ew_new = jnp.where(mask, ew, 0.0)
    return x_new, row_new, col_new, ew_new, batch_new, perm


def kernel(x, edge_index, batch, dW0, db0, dW1, db1, dW2, db2, dW3, db3,
           p0, p1, p2, uW0, ub0, uW1, ub1, uW2, ub2):
    dWs = [dW0, dW1, dW2, dW3]
    dbs = [db0, db1, db2, db3]
    ps = [p0, p1, p2]
    uWs = [uW0, uW1, uW2]
    ubs = [ub0, ub1, ub2]

    row = edge_index[0]
    col = edge_index[1]
    ew = jnp.where(row == col, 0.0, 1.0).astype(jnp.float32)
    sl = jnp.arange(N0, dtype=row.dtype)
    row = jnp.concatenate([row, sl])
    col = jnp.concatenate([col, sl])
    ew = jnp.concatenate([ew, jnp.ones((N0,), jnp.float32)])
    n = N0
    xc = jax.nn.relu(_gcn_ref(x, row, col, ew, dWs[0], dbs[0], n))
    xs = [xc]
    rows = [row]
    cols = [col]
    ews = [ew]
    ns = [n]
    perms = []
    for i in range(1, DEPTH + 1):
        k = int(math.ceil(RATIO * n))
        xc, row, col, ew, batch, perm = _pool(
            xc, row, col, ew, batch, ps[i - 1], n, k)
        n = k
        xc = jax.nn.relu(_gcn_ref(xc, row, col, ew, dWs[i], dbs[i], n))
        if i < DEPTH:
            xs.append(xc)
            rows.append(row)
            cols.append(col)
            ews.append(ew)
            ns.append(n)
        perms.append(perm)
    latent_x = xc
    latent_edge = jnp.stack([row, col])
    latent_batch = batch
    for i in range(DEPTH):
        j = DEPTH - 1 - i
        res = xs[j]
        up = jnp.zeros_like(res).at[perms[j]].set(xc)
        xc = res + up
        xc = _gcn_sc(xc, rows[j], cols[j], ews[j], uWs[i], ubs[i], ns[j])
        if i < DEPTH - 1:
            xc = jax.nn.relu(xc)
    return xc, latent_x, latent_edge, latent_batch
